# baseline (speedup 1.0000x reference)
import functools

import jax
import jax.numpy as jnp
from jax import lax
from jax.experimental import pallas as pl
from jax.experimental.pallas import tpu as pltpu
from jax.experimental.pallas import tpu_sc as plsc

B = 16384
D = 64
NEG = 5
NC = 2
NS = 16
NW = NC * NS
ROWS_PER_W = B // NW
C = 128
NCHUNK = ROWS_PER_W // C
L = 16
G = C // L


def _transe_body(h_hbm, r_hbm, t_hbm, tneg_hbm, ent_hbm, rel_hbm,
                 pos_hbm, neg_hbm,
                 hidx, ridx, tidx, nraw, nidx,
                 hbuf, rbuf, tbuf, nbuf,
                 pos_s, neg_s, sem):
    wid = lax.axis_index("s") * NC + lax.axis_index("c")
    wbase = wid * ROWS_PER_W

    def chunk_body(cc, carry):
        base = pl.multiple_of(wbase + cc * C, C)

        pltpu.sync_copy(h_hbm.at[pl.ds(base, C)], hidx)
        pltpu.sync_copy(r_hbm.at[pl.ds(base, C)], ridx)
        pltpu.sync_copy(t_hbm.at[pl.ds(base, C)], tidx)
        pltpu.sync_copy(tneg_hbm.at[pl.ds(base, C)], nraw)

        def regroup_body(g, carry2):
            rows = g * L + lax.iota(jnp.int32, L)
            for j in range(NEG):
                v = plsc.load_gather(nraw, [rows, jnp.full((L,), j, jnp.int32)])
                nidx[pl.ds(j * C + g * L, L)] = v
            return carry2

        lax.fori_loop(0, G, regroup_body, 0)

        cps = [
            pltpu.async_copy(ent_hbm.at[hidx], hbuf, sem),
            pltpu.async_copy(rel_hbm.at[ridx], rbuf, sem),
            pltpu.async_copy(ent_hbm.at[tidx], tbuf, sem),
        ]
        for j in range(NEG):
            cps.append(pltpu.async_copy(ent_hbm.at[nidx.at[pl.ds(j * C, C)]],
                                        nbuf.at[pl.ds(j * C, C)], sem))
        for cp in cps:
            cp.wait()

        def group_body(g, carry2):
            rows = g * L + lax.iota(jnp.int32, L)
            rows_n = [rows + j * C for j in range(NEG)]
            acc_p = jnp.zeros((L,), jnp.float32)
            acc_n = [jnp.zeros((L,), jnp.float32) for _ in range(NEG)]
            for d in range(D):
                col = jnp.full((L,), d, jnp.int32)
                hv = plsc.load_gather(hbuf, [rows, col])
                rv = plsc.load_gather(rbuf, [rows, col])
                tv = plsc.load_gather(tbuf, [rows, col])
                hr = hv + rv
                acc_p = acc_p + jnp.abs(hr - tv)
                for j in range(NEG):
                    nv = plsc.load_gather(nbuf, [rows_n[j], col])
                    acc_n[j] = acc_n[j] + jnp.abs(hr - nv)
            pos_s[pl.ds(g * L, L)] = acc_p
            for j in range(NEG):
                plsc.store_scatter(neg_s, [rows, jnp.full((L,), j, jnp.int32)],
                                   acc_n[j])
            return carry2

        lax.fori_loop(0, G, group_body, 0)

        pltpu.sync_copy(pos_s, pos_hbm.at[pl.ds(base, C)])
        pltpu.sync_copy(neg_s, neg_hbm.at[pl.ds(base, C)])
        return carry

    lax.fori_loop(0, NCHUNK, chunk_body, 0)


_transe_sc = functools.partial(
    pl.kernel,
    out_type=[
        jax.ShapeDtypeStruct((B,), jnp.float32),
        jax.ShapeDtypeStruct((B, NEG), jnp.float32),
    ],
    mesh=plsc.VectorSubcoreMesh(core_axis_name="c", subcore_axis_name="s"),
    compiler_params=pltpu.CompilerParams(needs_layout_passes=False,
                                         use_tc_tiling_on_sc=False),
    scratch_types=[
        pltpu.VMEM((C,), jnp.int32),
        pltpu.VMEM((C,), jnp.int32),
        pltpu.VMEM((C,), jnp.int32),
        pltpu.VMEM((C, NEG), jnp.int32),
        pltpu.VMEM((NEG * C,), jnp.int32),
        pltpu.VMEM((C, D), jnp.float32),
        pltpu.VMEM((C, D), jnp.float32),
        pltpu.VMEM((C, D), jnp.float32),
        pltpu.VMEM((NEG * C, D), jnp.float32),
        pltpu.VMEM((C,), jnp.float32),
        pltpu.VMEM((C, NEG), jnp.float32),
        pltpu.SemaphoreType.DMA,
    ],
)(_transe_body)


@jax.jit
def kernel(h, r, t, t_neg, entity_emb, relation_emb):
    h = h.astype(jnp.int32)
    r = r.astype(jnp.int32)
    t = t.astype(jnp.int32)
    t_neg = t_neg.astype(jnp.int32)
    pos, neg = _transe_sc(h, r, t, t_neg, entity_emb, relation_emb)
    return pos, neg


# speedup vs baseline: 1.0042x; 1.0042x over previous
import functools

import jax
import jax.numpy as jnp
from jax import lax
from jax.experimental import pallas as pl
from jax.experimental.pallas import tpu as pltpu
from jax.experimental.pallas import tpu_sc as plsc

B = 16384
D = 64
NEG = 5
NC = 2
NS = 16
NW = NC * NS
ROWS_PER_W = B // NW
C = 128
NCHUNK = ROWS_PER_W // C
L = 16
G = C // L
EV = 500000


def _transe_body(h_hbm, r_hbm, t_hbm, tneg_hbm, ent_hbm, rel_hbm,
                 pos_hbm, neg_hbm,
                 hidx, ridx, tidx, nraw, nidx,
                 hidx2, tidx2, nidx2,
                 hbuf, rbuf, tbuf, nbuf,
                 pos_s, neg_s, sem):
    wid = lax.axis_index("s") * NC + lax.axis_index("c")
    wbase = wid * ROWS_PER_W

    def chunk_body(cc, carry):
        base = pl.multiple_of(wbase + cc * C, C)

        pltpu.sync_copy(h_hbm.at[pl.ds(base, C)], hidx)
        pltpu.sync_copy(r_hbm.at[pl.ds(base, C)], ridx)
        pltpu.sync_copy(t_hbm.at[pl.ds(base, C)], tidx)
        pltpu.sync_copy(tneg_hbm.at[pl.ds(base, C)], nraw)

        def regroup_body(g, carry2):
            rows = g * L + lax.iota(jnp.int32, L)
            hv = hidx[pl.ds(g * L, L)]
            tv = tidx[pl.ds(g * L, L)]
            hidx2[pl.ds(g * L, L)] = jnp.right_shift(hv, 1)
            tidx2[pl.ds(g * L, L)] = jnp.right_shift(tv, 1)
            for j in range(NEG):
                v = plsc.load_gather(nraw, [rows, jnp.full((L,), j, jnp.int32)])
                nidx[pl.ds(j * C + g * L, L)] = v
                nidx2[pl.ds(j * C + g * L, L)] = jnp.right_shift(v, 1)
            return carry2

        lax.fori_loop(0, G, regroup_body, 0)

        cps = [
            pltpu.async_copy(ent_hbm.at[hidx2], hbuf, sem),
            pltpu.async_copy(rel_hbm.at[ridx], rbuf, sem),
            pltpu.async_copy(ent_hbm.at[tidx2], tbuf, sem),
        ]
        for j in range(NEG):
            cps.append(pltpu.async_copy(ent_hbm.at[nidx2.at[pl.ds(j * C, C)]],
                                        nbuf.at[pl.ds(j * C, C)], sem))
        for cp in cps:
            cp.wait()

        def group_body(g, carry2):
            rows = g * L + lax.iota(jnp.int32, L)
            rows_n = [rows + j * C for j in range(NEG)]
            half_h = jnp.left_shift(jnp.bitwise_and(hidx[pl.ds(g * L, L)], 1), 6)
            half_t = jnp.left_shift(jnp.bitwise_and(tidx[pl.ds(g * L, L)], 1), 6)
            half_n = [
                jnp.left_shift(
                    jnp.bitwise_and(nidx[pl.ds(j * C + g * L, L)], 1), 6)
                for j in range(NEG)
            ]
            acc_p = jnp.zeros((L,), jnp.float32)
            acc_n = [jnp.zeros((L,), jnp.float32) for _ in range(NEG)]
            for d in range(D):
                col = jnp.full((L,), d, jnp.int32)
                hv = plsc.load_gather(hbuf, [rows, half_h + d])
                rv = plsc.load_gather(rbuf, [rows, col])
                tv = plsc.load_gather(tbuf, [rows, half_t + d])
                hr = hv + rv
                acc_p = acc_p + jnp.abs(hr - tv)
                for j in range(NEG):
                    nv = plsc.load_gather(nbuf, [rows_n[j], half_n[j] + d])
                    acc_n[j] = acc_n[j] + jnp.abs(hr - nv)
            pos_s[pl.ds(g * L, L)] = acc_p
            for j in range(NEG):
                plsc.store_scatter(neg_s, [rows, jnp.full((L,), j, jnp.int32)],
                                   acc_n[j])
            return carry2

        lax.fori_loop(0, G, group_body, 0)

        pltpu.sync_copy(pos_s, pos_hbm.at[pl.ds(base, C)])
        pltpu.sync_copy(neg_s, neg_hbm.at[pl.ds(base, C)])
        return carry

    lax.fori_loop(0, NCHUNK, chunk_body, 0)


_transe_sc = functools.partial(
    pl.kernel,
    out_type=[
        jax.ShapeDtypeStruct((B,), jnp.float32),
        jax.ShapeDtypeStruct((B, NEG), jnp.float32),
    ],
    mesh=plsc.VectorSubcoreMesh(core_axis_name="c", subcore_axis_name="s"),
    compiler_params=pltpu.CompilerParams(needs_layout_passes=False,
                                         use_tc_tiling_on_sc=False),
    scratch_types=[
        pltpu.VMEM((C,), jnp.int32),
        pltpu.VMEM((C,), jnp.int32),
        pltpu.VMEM((C,), jnp.int32),
        pltpu.VMEM((C, NEG), jnp.int32),
        pltpu.VMEM((NEG * C,), jnp.int32),
        pltpu.VMEM((C,), jnp.int32),
        pltpu.VMEM((C,), jnp.int32),
        pltpu.VMEM((NEG * C,), jnp.int32),
        pltpu.VMEM((C, 2 * D), jnp.float32),
        pltpu.VMEM((C, D), jnp.float32),
        pltpu.VMEM((C, 2 * D), jnp.float32),
        pltpu.VMEM((NEG * C, 2 * D), jnp.float32),
        pltpu.VMEM((C,), jnp.float32),
        pltpu.VMEM((C, NEG), jnp.float32),
        pltpu.SemaphoreType.DMA,
    ],
)(_transe_body)


@jax.jit
def kernel(h, r, t, t_neg, entity_emb, relation_emb):
    h = h.astype(jnp.int32)
    r = r.astype(jnp.int32)
    t = t.astype(jnp.int32)
    t_neg = t_neg.astype(jnp.int32)
    ent2 = jnp.reshape(entity_emb, (EV, 2 * D))
    pos, neg = _transe_sc(h, r, t, t_neg, ent2, relation_emb)
    return pos, neg
